# Initial kernel scaffold; baseline (speedup 1.0000x reference)
import jax
import jax.numpy as jnp
from jax.experimental import pallas as pl


def kernel(x, z, per_item, x2y_w, z2y_w, y2z_w, y_neuron_age):
    raise NotImplementedError("write your pallas kernel here")


# speedup vs baseline: 1.0559x; 1.0559x over previous
import jax
import jax.numpy as jnp
from jax.experimental import pallas as pl
from jax.experimental.pallas import tpu as pltpu

_Y_CHUNK = 8192


def _split_bf16(a):
    hi = a.astype(jnp.bfloat16)
    lo = (a - hi.astype(jnp.float32)).astype(jnp.bfloat16)
    return hi, lo


def _dotn(a, b):
    return jax.lax.dot_general(a, b, (((1,), (1,)), ((), ())),
                               preferred_element_type=jnp.float32)


def _dn_step(x_ref, w_ref, age_ref, y2z_ref, out_ref, max_ref, cand_ref, ssq_ref):
    i = pl.program_id(0)
    nsteps = pl.num_programs(0)

    @pl.when(i == 0)
    def _init():
        max_ref[...] = jnp.full_like(max_ref, -jnp.inf)
        cand_ref[...] = jnp.zeros_like(cand_ref)
        ssq_ref[...] = jnp.zeros_like(ssq_ref)

    xs = x_ref[...]
    w = w_ref[...]
    y2z = y2z_ref[...]
    age = age_ref[...]
    chunk = w.shape[0]
    b = xs.shape[0] // 2

    wh, wl = _split_bf16(w)
    p1 = _dotn(xs, wh)
    p2 = _dotn(xs, wl)
    dots = (p1[:b] + p1[b:]) + (p2[:b] + p2[b:])

    ones_x = jnp.ones((1, w.shape[1]), jnp.bfloat16)
    sqh, sql = _split_bf16(w * w)
    wssq = _dotn(ones_x, sqh) + _dotn(ones_x, sql)

    recip = 1.0 / jnp.maximum(jnp.sqrt(wssq), 1e-12)
    act = jnp.where(age >= 1.0, 1.0, 0.0)
    y_pre = dots * (recip * act)

    local_max = jnp.max(y_pre, axis=1, keepdims=True)
    iota = jax.lax.broadcasted_iota(jnp.int32, y_pre.shape, 1)
    eq = y_pre == local_max
    first = jnp.min(jnp.where(eq, iota, chunk), axis=1, keepdims=True)
    onehot = (iota == first).astype(jnp.float32)

    cand = _dotn(onehot, y2z)
    better = local_max > max_ref[...]
    max_ref[...] = jnp.where(better, local_max, max_ref[...])
    cand_ref[...] = jnp.where(better, cand, cand_ref[...])

    ones_y = jnp.ones((1, chunk), jnp.float32)
    ssq_ref[...] += _dotn(ones_y, y2z * y2z)

    @pl.when(i == nsteps - 1)
    def _fin():
        zn = jnp.maximum(jnp.sqrt(ssq_ref[...]), 1e-12)
        out_ref[...] = cand_ref[...] / zn


def kernel(x, z, per_item, x2y_w, z2y_w, y2z_w, y_neuron_age):
    batch = x.shape[0]
    xf = x.reshape(batch, -1)
    x_dim = xf.shape[1]
    y_num = x2y_w.shape[0]
    z_num = y2z_w.shape[0]
    grid = y_num // _Y_CHUNK

    xh = xf.astype(jnp.bfloat16)
    xl = (xf - xh.astype(jnp.float32)).astype(jnp.bfloat16)
    xs = jnp.concatenate([xh, xl], axis=0)

    return pl.pallas_call(
        _dn_step,
        grid=(grid,),
        in_specs=[
            pl.BlockSpec((2 * batch, x_dim), lambda i: (0, 0)),
            pl.BlockSpec((_Y_CHUNK, x_dim), lambda i: (i, 0)),
            pl.BlockSpec((1, _Y_CHUNK), lambda i: (0, i)),
            pl.BlockSpec((z_num, _Y_CHUNK), lambda i: (0, i)),
        ],
        out_specs=pl.BlockSpec((batch, z_num), lambda i: (0, 0)),
        out_shape=jax.ShapeDtypeStruct((batch, z_num), jnp.float32),
        scratch_shapes=[
            pltpu.VMEM((batch, 1), jnp.float32),
            pltpu.VMEM((batch, z_num), jnp.float32),
            pltpu.VMEM((1, z_num), jnp.float32),
        ],
    )(xs, x2y_w, y_neuron_age, y2z_w)


# speedup vs baseline: 1.0845x; 1.0271x over previous
import jax
import jax.numpy as jnp
from jax.experimental import pallas as pl
from jax.experimental.pallas import tpu as pltpu

_Y_CHUNK = 4096


def _split_bf16(a):
    hi = a.astype(jnp.bfloat16)
    lo = (a - hi.astype(jnp.float32)).astype(jnp.bfloat16)
    return hi, lo


def _dotn(a, b):
    return jax.lax.dot_general(a, b, (((1,), (1,)), ((), ())),
                               preferred_element_type=jnp.float32)


def _dn_step(x_ref, w_ref, age_ref, y2z_ref, out_ref, max_ref, cand_ref, ssq_ref):
    i = pl.program_id(0)
    nsteps = pl.num_programs(0)

    @pl.when(i == 0)
    def _init():
        max_ref[...] = jnp.full_like(max_ref, -jnp.inf)
        cand_ref[...] = jnp.zeros_like(cand_ref)
        ssq_ref[...] = jnp.zeros_like(ssq_ref)

    xs = x_ref[...]
    w = w_ref[...]
    y2z = y2z_ref[...]
    age = age_ref[...]
    chunk = w.shape[0]
    b = xs.shape[0] // 2

    wh, wl = _split_bf16(w)
    p1 = _dotn(xs, wh)
    p2 = _dotn(xs, wl)
    dots = (p1[:b] + p1[b:]) + (p2[:b] + p2[b:])

    ones_x = jnp.ones((1, w.shape[1]), jnp.bfloat16)
    sqh, sql = _split_bf16(w * w)
    wssq = _dotn(ones_x, sqh) + _dotn(ones_x, sql)

    recip = 1.0 / jnp.maximum(jnp.sqrt(wssq), 1e-12)
    act = jnp.where(age >= 1.0, 1.0, 0.0)
    y_pre = dots * (recip * act)

    local_max = jnp.max(y_pre, axis=1, keepdims=True)
    iota = jax.lax.broadcasted_iota(jnp.int32, y_pre.shape, 1)
    eq = y_pre == local_max
    first = jnp.min(jnp.where(eq, iota, chunk), axis=1, keepdims=True)
    onehot = (iota == first).astype(jnp.float32)

    cand = _dotn(onehot, y2z)
    better = local_max > max_ref[...]
    max_ref[...] = jnp.where(better, local_max, max_ref[...])
    cand_ref[...] = jnp.where(better, cand, cand_ref[...])

    ones_y = jnp.ones((1, chunk), jnp.float32)
    ssq_ref[...] += _dotn(ones_y, y2z * y2z)

    @pl.when(i == nsteps - 1)
    def _fin():
        zn = jnp.maximum(jnp.sqrt(ssq_ref[...]), 1e-12)
        out_ref[...] = cand_ref[...] / zn


def kernel(x, z, per_item, x2y_w, z2y_w, y2z_w, y_neuron_age):
    batch = x.shape[0]
    xf = x.reshape(batch, -1)
    x_dim = xf.shape[1]
    y_num = x2y_w.shape[0]
    z_num = y2z_w.shape[0]
    grid = y_num // _Y_CHUNK

    xh = xf.astype(jnp.bfloat16)
    xl = (xf - xh.astype(jnp.float32)).astype(jnp.bfloat16)
    xs = jnp.concatenate([xh, xl], axis=0)

    return pl.pallas_call(
        _dn_step,
        grid=(grid,),
        in_specs=[
            pl.BlockSpec((2 * batch, x_dim), lambda i: (0, 0)),
            pl.BlockSpec((_Y_CHUNK, x_dim), lambda i: (i, 0)),
            pl.BlockSpec((1, _Y_CHUNK), lambda i: (0, i)),
            pl.BlockSpec((z_num, _Y_CHUNK), lambda i: (0, i)),
        ],
        out_specs=pl.BlockSpec((batch, z_num), lambda i: (0, 0)),
        out_shape=jax.ShapeDtypeStruct((batch, z_num), jnp.float32),
        scratch_shapes=[
            pltpu.VMEM((batch, 1), jnp.float32),
            pltpu.VMEM((batch, z_num), jnp.float32),
            pltpu.VMEM((1, z_num), jnp.float32),
        ],
    )(xs, x2y_w, y_neuron_age, y2z_w)
